# baseline (speedup 1.0000x reference)
import jax
import jax.numpy as jnp
from jax import lax
from jax.experimental import pallas as pl
from jax.experimental.pallas import tpu as pltpu
from jax.experimental.pallas import tpu_sc as plsc

_B = 16
_TOTAL = 32768
_L = 2048
_LANES = 16
_CHUNK = _L
_STAGE = _CHUNK + 8
_IW = 128
_NSTREAM = _CHUNK // _IW
_NCH = _CHUNK // _LANES


def _tok_body(flat_hbm, cu_hbm, table_hbm, out_hbm, cu_v, ft_v, tok_v, val_v, sem):
    b = lax.axis_index("s")

    pltpu.sync_copy(cu_hbm, cu_v)

    lane = lax.iota(jnp.int32, _LANES)
    b_splat = jnp.full((_LANES,), b, jnp.int32)
    start = jnp.max(plsc.load_gather(cu_v, [b_splat]))
    end_splat = plsc.load_gather(cu_v, [b_splat + 1])
    end = jnp.max(end_splat)

    off_raw = start
    n_valid = jnp.clip(end - off_raw, 0, _CHUNK)

    off = pl.multiple_of(jnp.minimum(off_raw & -8, _TOTAL - _STAGE), 8)
    base = off_raw - off

    @pl.when(n_valid > 0)
    def _stage():
        pltpu.sync_copy(flat_hbm.at[pl.ds(off, _STAGE)], ft_v)

    copies = []
    for r in range(_NSTREAM):
        fire = n_valid > r * _IW

        @pl.when(fire)
        def _prep_and_fire():
            @plsc.parallel_loop(r * (_IW // _LANES), (r + 1) * (_IW // _LANES),
                                unroll=4)
            def prep(j):
                lpos = j * _LANES + lane
                lidx = jnp.clip(base + lpos, 0, _STAGE - 1)
                tok = plsc.load_gather(ft_v, [lidx])
                plsc.store_scatter(tok_v, [lpos], tok)

            pltpu.async_copy(
                table_hbm.at[tok_v.at[pl.ds(r * _IW, _IW)]],
                val_v.at[pl.ds(r * _IW, _IW)],
                sem,
            )

        copies.append(fire)
    for r, fire in enumerate(copies):
        @pl.when(fire)
        def _drain():
            pltpu.make_async_copy(
                table_hbm.at[tok_v.at[pl.ds(r * _IW, _IW)]],
                val_v.at[pl.ds(r * _IW, _IW)],
                sem,
            ).wait()

    jz = n_valid // _LANES

    @pl.when(jz < _NCH)
    def _mixed():
        lpos = jz * _LANES + lane
        val = plsc.load_gather(val_v, [lpos])
        keep = (off_raw + lpos) < end_splat
        plsc.store_scatter(val_v, [lpos], jnp.where(keep, val, 0.0))

    zeros = jnp.zeros((_LANES,), jnp.float32)

    def zfill(j, c):
        plsc.store_scatter(val_v, [j * _LANES + lane], zeros)
        return c

    lax.fori_loop(jz + 1, _NCH, zfill, 0)

    pltpu.sync_copy(val_v, out_hbm.at[b])


def kernel(flat_tokens, cu_seqlens, lookup_table):
    mesh = plsc.VectorSubcoreMesh(
        core_axis_name="c", subcore_axis_name="s", num_cores=1
    )
    out = pl.kernel(
        _tok_body,
        out_type=jax.ShapeDtypeStruct((_B, _L), jnp.float32),
        mesh=mesh,
        compiler_params=pltpu.CompilerParams(
            needs_layout_passes=False, use_tc_tiling_on_sc=False
        ),
        scratch_types=[
            pltpu.VMEM((_B + 1,), jnp.int32),
            pltpu.VMEM((_STAGE,), jnp.int32),
            pltpu.VMEM((_CHUNK,), jnp.int32),
            pltpu.VMEM((_CHUNK,), jnp.float32),
            pltpu.SemaphoreType.DMA,
        ],
    )(flat_tokens, cu_seqlens.astype(jnp.int32), lookup_table)
    return out


# speedup vs baseline: 1.0370x; 1.0370x over previous
import jax
import jax.numpy as jnp
from jax import lax
from jax.experimental import pallas as pl
from jax.experimental.pallas import tpu as pltpu
from jax.experimental.pallas import tpu_sc as plsc

_B = 16
_TOTAL = 32768
_L = 2048
_LANES = 16
_NC = 2
_NS = 16
_NW = _NC * _NS
_CHUNK = (_B * _L) // _NW
_STAGE = _CHUNK + 8
_NSTREAM = 8
_IW = _CHUNK // _NSTREAM
_NCH = _CHUNK // _LANES


def _tok_body(flat_hbm, cu_hbm, table_hbm, out_hbm, cu_v, ft_v, tok_v, val_v,
              sem, osem):
    wid = lax.axis_index("s") * _NC + lax.axis_index("c")
    b = wid // 2
    h = wid % 2
    l0 = h * _CHUNK

    pltpu.sync_copy(cu_hbm, cu_v)

    lane = lax.iota(jnp.int32, _LANES)
    b_splat = jnp.full((_LANES,), b, jnp.int32)
    start = jnp.max(plsc.load_gather(cu_v, [b_splat]))
    end_splat = plsc.load_gather(cu_v, [b_splat + 1])
    end = jnp.max(end_splat)

    off_raw = start + l0
    n_valid = jnp.clip(end - off_raw, 0, _CHUNK)

    off = pl.multiple_of(jnp.minimum(off_raw & -8, _TOTAL - _STAGE), 8)
    base = off_raw - off

    @pl.when(n_valid > 0)
    def _stage():
        pltpu.sync_copy(flat_hbm.at[pl.ds(off, _STAGE)], ft_v)

    copies = []
    for r in range(_NSTREAM):
        fire = n_valid > r * _IW

        @pl.when(fire)
        def _prep_and_fire():
            @plsc.parallel_loop(r * (_IW // _LANES), (r + 1) * (_IW // _LANES),
                                unroll=4)
            def prep(j):
                lpos = j * _LANES + lane
                lidx = jnp.clip(base + lpos, 0, _STAGE - 1)
                tok = plsc.load_gather(ft_v, [lidx])
                plsc.store_scatter(tok_v, [lpos], tok)

            pltpu.async_copy(
                table_hbm.at[tok_v.at[pl.ds(r * _IW, _IW)]],
                val_v.at[pl.ds(r * _IW, _IW)],
                sem,
            )

        copies.append(fire)

    jz = n_valid // _LANES
    zeros = jnp.zeros((_LANES,), jnp.float32)

    def zfill(j, c):
        plsc.store_scatter(val_v, [j * _LANES + lane], zeros)
        return c

    hw = _CHUNK // 2
    hch = _NCH // 2
    for half in range(2):
        for r in range(half * _NSTREAM // 2, (half + 1) * _NSTREAM // 2):
            @pl.when(copies[r])
            def _drain():
                pltpu.make_async_copy(
                    table_hbm.at[tok_v.at[pl.ds(r * _IW, _IW)]],
                    val_v.at[pl.ds(r * _IW, _IW)],
                    sem,
                ).wait()

        @pl.when((jz >= half * hch) & (jz < (half + 1) * hch))
        def _mixed():
            lpos = jz * _LANES + lane
            val = plsc.load_gather(val_v, [lpos])
            keep = (off_raw + lpos) < end_splat
            plsc.store_scatter(val_v, [lpos], jnp.where(keep, val, 0.0))

        lax.fori_loop(jnp.clip(jz + 1, half * hch, (half + 1) * hch),
                      (half + 1) * hch, zfill, 0)

        pltpu.async_copy(
            val_v.at[pl.ds(half * hw, hw)],
            out_hbm.at[b, h, pl.ds(half * hw, hw)],
            osem,
        )

    for half in range(2):
        pltpu.make_async_copy(
            val_v.at[pl.ds(half * hw, hw)],
            out_hbm.at[b, h, pl.ds(half * hw, hw)],
            osem,
        ).wait()


def kernel(flat_tokens, cu_seqlens, lookup_table):
    mesh = plsc.VectorSubcoreMesh(core_axis_name="c", subcore_axis_name="s")
    out = pl.kernel(
        _tok_body,
        out_type=jax.ShapeDtypeStruct((_B, 2, _CHUNK), jnp.float32),
        mesh=mesh,
        compiler_params=pltpu.CompilerParams(
            needs_layout_passes=False, use_tc_tiling_on_sc=False
        ),
        scratch_types=[
            pltpu.VMEM((_B + 1,), jnp.int32),
            pltpu.VMEM((_STAGE,), jnp.int32),
            pltpu.VMEM((_CHUNK,), jnp.int32),
            pltpu.VMEM((_CHUNK,), jnp.float32),
            pltpu.SemaphoreType.DMA,
            pltpu.SemaphoreType.DMA,
        ],
    )(flat_tokens, cu_seqlens.astype(jnp.int32), lookup_table)
    return out.reshape(_B, _L)


# speedup vs baseline: 1.0527x; 1.0151x over previous
import jax
import jax.numpy as jnp
from jax import lax
from jax.experimental import pallas as pl
from jax.experimental.pallas import tpu as pltpu
from jax.experimental.pallas import tpu_sc as plsc

_B = 16
_TOTAL = 32768
_L = 2048
_LANES = 16
_NC = 2
_NS = 16
_NW = _NC * _NS
_CHUNK = (_B * _L) // _NW
_STAGE = _CHUNK + 8
_NSTREAM = 8
_IW = _CHUNK // _NSTREAM
_NCH = _CHUNK // _LANES


def _tok_body(flat_hbm, cu_hbm, table_hbm, out_hbm, cu_v, ft_v, tok_v, val_v, sem):
    wid = lax.axis_index("s") * _NC + lax.axis_index("c")
    b = wid // 2
    h = wid % 2
    l0 = h * _CHUNK

    pltpu.sync_copy(cu_hbm, cu_v)

    lane = lax.iota(jnp.int32, _LANES)
    b_splat = jnp.full((_LANES,), b, jnp.int32)
    start = jnp.max(plsc.load_gather(cu_v, [b_splat]))
    end_splat = plsc.load_gather(cu_v, [b_splat + 1])
    end = jnp.max(end_splat)

    off_raw = start + l0
    n_valid = jnp.clip(end - off_raw, 0, _CHUNK)

    off = pl.multiple_of(jnp.minimum(off_raw & -8, _TOTAL - _STAGE), 8)
    base = off_raw - off

    @pl.when(n_valid > 0)
    def _stage():
        pltpu.sync_copy(flat_hbm.at[pl.ds(off, _STAGE)], ft_v)

    copies = []
    for r in range(_NSTREAM):
        fire = n_valid > r * _IW

        @pl.when(fire)
        def _prep_and_fire():
            @plsc.parallel_loop(r * (_IW // _LANES), (r + 1) * (_IW // _LANES),
                                unroll=8)
            def prep(j):
                lpos = j * _LANES + lane
                lidx = jnp.clip(base + lpos, 0, _STAGE - 1)
                tok = plsc.load_gather(ft_v, [lidx])
                plsc.store_scatter(tok_v, [lpos], tok)

            pltpu.async_copy(
                table_hbm.at[tok_v.at[pl.ds(r * _IW, _IW)]],
                val_v.at[pl.ds(r * _IW, _IW)],
                sem,
            )

        copies.append(fire)
    for r, fire in enumerate(copies):
        @pl.when(fire)
        def _drain():
            pltpu.make_async_copy(
                table_hbm.at[tok_v.at[pl.ds(r * _IW, _IW)]],
                val_v.at[pl.ds(r * _IW, _IW)],
                sem,
            ).wait()

    jz = n_valid // _LANES

    @pl.when(jz < _NCH)
    def _mixed():
        lpos = jz * _LANES + lane
        val = plsc.load_gather(val_v, [lpos])
        keep = (off_raw + lpos) < end_splat
        plsc.store_scatter(val_v, [lpos], jnp.where(keep, val, 0.0))

    zeros = jnp.zeros((_LANES,), jnp.float32)

    def zfill(j, c):
        plsc.store_scatter(val_v, [j * _LANES + lane], zeros)
        return c

    lax.fori_loop(jz + 1, _NCH, zfill, 0)

    pltpu.sync_copy(val_v, out_hbm.at[b, h])


def kernel(flat_tokens, cu_seqlens, lookup_table):
    mesh = plsc.VectorSubcoreMesh(core_axis_name="c", subcore_axis_name="s")
    out = pl.kernel(
        _tok_body,
        out_type=jax.ShapeDtypeStruct((_B, 2, _CHUNK), jnp.float32),
        mesh=mesh,
        compiler_params=pltpu.CompilerParams(
            needs_layout_passes=False, use_tc_tiling_on_sc=False
        ),
        scratch_types=[
            pltpu.VMEM((_B + 1,), jnp.int32),
            pltpu.VMEM((_STAGE,), jnp.int32),
            pltpu.VMEM((_CHUNK,), jnp.int32),
            pltpu.VMEM((_CHUNK,), jnp.float32),
            pltpu.SemaphoreType.DMA,
        ],
    )(flat_tokens, cu_seqlens.astype(jnp.int32), lookup_table)
    return out.reshape(_B, _L)


# speedup vs baseline: 1.0539x; 1.0012x over previous
import jax
import jax.numpy as jnp
from jax import lax
from jax.experimental import pallas as pl
from jax.experimental.pallas import tpu as pltpu
from jax.experimental.pallas import tpu_sc as plsc

_B = 16
_TOTAL = 32768
_L = 2048
_LANES = 16
_NC = 2
_NS = 16
_NW = _NC * _NS
_CHUNK = (_B * _L) // _NW
_STAGE = _CHUNK + 8
_NSTREAM = 8
_IW = _CHUNK // _NSTREAM
_NCH = _CHUNK // _LANES


def _tok_body(flat_hbm, cu_hbm, table_hbm, out_hbm, cu_v, ft_v, tok_v, val_v, sem):
    wid = lax.axis_index("s") * _NC + lax.axis_index("c")
    b = wid // 2
    h = wid % 2
    l0 = h * _CHUNK

    pltpu.sync_copy(cu_hbm, cu_v)

    lane = lax.iota(jnp.int32, _LANES)
    b_splat = jnp.full((_LANES,), b, jnp.int32)
    start = jnp.max(plsc.load_gather(cu_v, [b_splat]))
    end_splat = plsc.load_gather(cu_v, [b_splat + 1])
    end = jnp.max(end_splat)

    off_raw = start + l0
    n_valid = jnp.clip(end - off_raw, 0, _CHUNK)

    off = pl.multiple_of(jnp.minimum(off_raw & -8, _TOTAL - _STAGE), 8)
    base = off_raw - off

    @pl.when(n_valid > 0)
    def _stage():
        pltpu.sync_copy(flat_hbm.at[pl.ds(off, _STAGE)], ft_v)

    copies = []
    for r in range(_NSTREAM):
        fire = n_valid > r * _IW

        @pl.when(fire)
        def _prep_and_fire():
            @plsc.parallel_loop(r * (_IW // _LANES), (r + 1) * (_IW // _LANES),
                                unroll=8)
            def prep(j):
                lpos = j * _LANES + lane
                lidx = jnp.minimum(base + lpos, _STAGE - 1)
                tok = plsc.load_gather(ft_v, [lidx])
                plsc.store_scatter(tok_v, [lpos], tok)

            pltpu.async_copy(
                table_hbm.at[tok_v.at[pl.ds(r * _IW, _IW)]],
                val_v.at[pl.ds(r * _IW, _IW)],
                sem,
            )

        copies.append(fire)
    for r, fire in enumerate(copies):
        @pl.when(fire)
        def _drain():
            pltpu.make_async_copy(
                table_hbm.at[tok_v.at[pl.ds(r * _IW, _IW)]],
                val_v.at[pl.ds(r * _IW, _IW)],
                sem,
            ).wait()

    jz = n_valid // _LANES

    @pl.when(jz < _NCH)
    def _mixed():
        lpos = jz * _LANES + lane
        val = plsc.load_gather(val_v, [lpos])
        keep = (off_raw + lpos) < end_splat
        plsc.store_scatter(val_v, [lpos], jnp.where(keep, val, 0.0))

    zeros = jnp.zeros((_LANES,), jnp.float32)

    def zfill(j, c):
        plsc.store_scatter(val_v, [j * _LANES + lane], zeros)
        return c

    lax.fori_loop(jz + 1, _NCH, zfill, 0)

    pltpu.sync_copy(val_v, out_hbm.at[b, h])


def kernel(flat_tokens, cu_seqlens, lookup_table):
    mesh = plsc.VectorSubcoreMesh(core_axis_name="c", subcore_axis_name="s")
    out = pl.kernel(
        _tok_body,
        out_type=jax.ShapeDtypeStruct((_B, 2, _CHUNK), jnp.float32),
        mesh=mesh,
        compiler_params=pltpu.CompilerParams(
            needs_layout_passes=False, use_tc_tiling_on_sc=False
        ),
        scratch_types=[
            pltpu.VMEM((_B + 1,), jnp.int32),
            pltpu.VMEM((_STAGE,), jnp.int32),
            pltpu.VMEM((_CHUNK,), jnp.int32),
            pltpu.VMEM((_CHUNK,), jnp.float32),
            pltpu.SemaphoreType.DMA,
        ],
    )(flat_tokens, cu_seqlens.astype(jnp.int32), lookup_table)
    return out.reshape(_B, _L)
